# Initial kernel scaffold; baseline (speedup 1.0000x reference)
import jax
import jax.numpy as jnp
from jax.experimental import pallas as pl


def kernel(activation):
    raise NotImplementedError("write your pallas kernel here")


# speedup vs baseline: 5.0107x; 5.0107x over previous
import functools

import jax
import jax.numpy as jnp
from jax import lax
from jax.experimental import pallas as pl
from jax.experimental.pallas import tpu as pltpu
from jax.experimental.pallas import tpu_sc as plsc

C, H, W = 96, 512, 512
NC, NS = 2, 16
NW = NC * NS
RPW = H // NW
LG = W // 16

_DN = lax.GatherDimensionNumbers(
    offset_dims=(), collapsed_slice_dims=(0,), start_index_map=(0,))


def _perm(v, idx2d):
    return lax.gather(v, idx2d, dimension_numbers=_DN, slice_sizes=(1,),
                      mode=lax.GatherScatterMode.PROMISE_IN_BOUNDS)


def _make_kernel():
    mesh = plsc.VectorSubcoreMesh(core_axis_name="c", subcore_axis_name="s")

    @functools.partial(
        pl.kernel,
        out_type=jax.ShapeDtypeStruct((C, H, W), jnp.float32),
        mesh=mesh,
        scratch_types=[pltpu.VMEM((RPW, W), jnp.float32)],
    )
    def k(act, out, buf):
        wid = lax.axis_index("s") * NC + lax.axis_index("c")
        r0 = wid * RPW
        iot = lax.iota(jnp.int32, 16)
        p1 = (iot ^ 1)[:, None]
        p2 = (iot ^ 2)[:, None]
        zero = jnp.zeros((16,), jnp.float32)

        def block2(col):
            for p in range(RPW // 2):
                a = buf[2 * p, pl.ds(col, 16)]
                b = buf[2 * p + 1, pl.ds(col, 16)]
                r = a + b
                s = r + _perm(r, p1)
                m = s > 0.0
                buf[2 * p, pl.ds(col, 16)] = jnp.where(m, a, zero)
                buf[2 * p + 1, pl.ds(col, 16)] = jnp.where(m, b, zero)

        def block4(col):
            for q in range(RPW // 4):
                vs = [buf[4 * q + i, pl.ds(col, 16)] for i in range(4)]
                r = (vs[0] + vs[1]) + (vs[2] + vs[3])
                s2 = r + _perm(r, p1)
                s4 = s2 + _perm(s2, p2)
                m = s4 > 0.0
                for i in range(4):
                    buf[4 * q + i, pl.ds(col, 16)] = jnp.where(m, vs[i], zero)

        def relu(col):
            for rr in range(RPW):
                v = buf[rr, pl.ds(col, 16)]
                buf[rr, pl.ds(col, 16)] = jnp.maximum(v, 0.0)

        def run_group(c_lo, c_hi, compute):
            def chan_body(ch, carry):
                pltpu.sync_copy(act.at[ch, pl.ds(r0, RPW)], buf)
                if compute is not None:
                    def col_body(j, c2):
                        compute(j * 16)
                        return c2
                    lax.fori_loop(0, LG, col_body, 0)
                pltpu.sync_copy(buf, out.at[ch, pl.ds(r0, RPW)])
                return carry
            lax.fori_loop(c_lo, c_hi, chan_body, 0)

        run_group(0, 32, block2)
        run_group(32, 64, block4)
        run_group(64, 80, relu)
        run_group(80, 96, None)

    return k


_k = _make_kernel()


def kernel(activation):
    act3 = activation.reshape(C, H, W)
    out = _k(act3)
    return out.reshape(1, C, H, W)


# speedup vs baseline: 5.6796x; 1.1335x over previous
import functools

import jax
import jax.numpy as jnp
from jax import lax
from jax.experimental import pallas as pl
from jax.experimental.pallas import tpu as pltpu
from jax.experimental.pallas import tpu_sc as plsc

C, H, W = 96, 512, 512
NC, NS = 2, 16
NW = NC * NS
RPW = H // NW
LG = W // 16

_DN = lax.GatherDimensionNumbers(
    offset_dims=(), collapsed_slice_dims=(0,), start_index_map=(0,))


def _perm(v, idx2d):
    return lax.gather(v, idx2d, dimension_numbers=_DN, slice_sizes=(1,),
                      mode=lax.GatherScatterMode.PROMISE_IN_BOUNDS)


def _make_kernel():
    mesh = plsc.VectorSubcoreMesh(core_axis_name="c", subcore_axis_name="s")

    @functools.partial(
        pl.kernel,
        out_type=jax.ShapeDtypeStruct((C, H, W), jnp.float32),
        mesh=mesh,
        scratch_types=[
            pltpu.VMEM((2, RPW, W), jnp.float32),
            pltpu.VMEM((2, RPW, W), jnp.float32),
            pltpu.SemaphoreType.DMA,
            pltpu.SemaphoreType.DMA,
            pltpu.SemaphoreType.DMA,
            pltpu.SemaphoreType.DMA,
        ],
    )
    def k(act, out, ib, ob, si0, si1, so0, so1):
        wid = lax.axis_index("s") * NC + lax.axis_index("c")
        r0 = wid * RPW
        sem_in = (si0, si1)
        sem_out = (so0, so1)
        iot = lax.iota(jnp.int32, 16)
        p1 = (iot ^ 1)[:, None]
        p2 = (iot ^ 2)[:, None]
        zero = jnp.zeros((16,), jnp.float32)

        def in_copy(ch, b):
            return pltpu.make_async_copy(
                act.at[ch, pl.ds(r0, RPW)], ib.at[b], sem_in[b])

        def out_copy(ch, b):
            return pltpu.make_async_copy(
                ob.at[b], out.at[ch, pl.ds(r0, RPW)], sem_out[b])

        def block2(b, col):
            for p in range(RPW // 2):
                a = ib[b, 2 * p, pl.ds(col, 16)]
                c = ib[b, 2 * p + 1, pl.ds(col, 16)]
                r = a + c
                s = r + _perm(r, p1)
                m = s > 0.0
                ob[b, 2 * p, pl.ds(col, 16)] = jnp.where(m, a, zero)
                ob[b, 2 * p + 1, pl.ds(col, 16)] = jnp.where(m, c, zero)

        def block4(b, col):
            for q in range(RPW // 4):
                vs = [ib[b, 4 * q + i, pl.ds(col, 16)] for i in range(4)]
                r = (vs[0] + vs[1]) + (vs[2] + vs[3])
                s2 = r + _perm(r, p1)
                s4 = s2 + _perm(s2, p2)
                m = s4 > 0.0
                for i in range(4):
                    ob[b, 4 * q + i, pl.ds(col, 16)] = jnp.where(m, vs[i], zero)

        def relu(b, col):
            for rr in range(RPW):
                v = ib[b, rr, pl.ds(col, 16)]
                ob[b, rr, pl.ds(col, 16)] = jnp.maximum(v, 0.0)

        def ident(b, col):
            for rr in range(RPW):
                ob[b, rr, pl.ds(col, 16)] = ib[b, rr, pl.ds(col, 16)]

        def section(lo, n, compute):
            n2 = n // 2
            for b in range(2):
                in_copy(lo + b, b).start()

            def body(i, carry):
                for b in range(2):
                    ch = lo + 2 * i + b
                    in_copy(ch, b).wait()

                    @pl.when(i >= 1)
                    def _wait_prev_out():
                        out_copy(ch, b).wait()

                    def col_body(j, c2):
                        compute(b, j * 16)
                        return c2
                    lax.fori_loop(0, LG, col_body, 0)
                    out_copy(ch, b).start()

                    @pl.when(i < n2 - 1)
                    def _start_next_in():
                        in_copy(ch + 2, b).start()
                return carry

            lax.fori_loop(0, n2, body, 0)
            for b in range(2):
                out_copy(lo + b, b).wait()

        section(0, 32, block2)
        section(32, 32, block4)
        section(64, 16, relu)
        section(80, 16, ident)

    return k


_k = _make_kernel()


def kernel(activation):
    act3 = activation.reshape(C, H, W)
    out = _k(act3)
    return out.reshape(1, C, H, W)
